# baseline (speedup 1.0000x reference)
import functools

import jax
import jax.numpy as jnp
from jax import lax
from jax.experimental import pallas as pl
from jax.experimental.pallas import tpu as pltpu
from jax.experimental.pallas import tpu_sc as plsc

_B, _N, _D = 16, 512, 16
_NC, _NS = 2, 16
_NW = _NC * _NS
_R = _B * _N
_RPW = _R // _NW
_CH = 128
_LOG2_N = 9
_LOG2_NN = 18


def _sc_row_ids(idx_ref, start):
    for t in range(_CH // 16):
        g = start + t * 16 + lax.broadcasted_iota(jnp.int32, (16,), 0)
        b = g >> _LOG2_N
        i = g & (_N - 1)
        idx_ref[pl.ds(t * 16, 16)] = (b << _LOG2_NN) + i * (_N + 1)


def _sc_gather_body(table_hbm, out_hbm, idx0, idx1, rows, sem):
    wid = lax.axis_index("s") * _NC + lax.axis_index("c")
    base = wid * _RPW
    _sc_row_ids(idx0, base)
    _sc_row_ids(idx1, base + _CH)
    cp0 = pltpu.async_copy(table_hbm.at[idx0], rows.at[pl.ds(0, _CH)], sem)
    cp1 = pltpu.async_copy(table_hbm.at[idx1], rows.at[pl.ds(_CH, _CH)], sem)
    cp0.wait()
    cp1.wait()
    pltpu.sync_copy(rows, out_hbm.at[pl.ds(base, _RPW)])


@functools.cache
def _sc_gather():
    return pl.kernel(
        _sc_gather_body,
        out_type=jax.ShapeDtypeStruct((_R, _D), jnp.float32),
        mesh=plsc.VectorSubcoreMesh(
            core_axis_name="c", subcore_axis_name="s",
            num_cores=_NC, num_subcores=_NS,
        ),
        scratch_types=[
            pltpu.VMEM((_CH,), jnp.int32),
            pltpu.VMEM((_CH,), jnp.int32),
            pltpu.VMEM((_RPW, _D), jnp.float32),
            pltpu.SemaphoreType.DMA,
        ],
        compiler_params=pltpu.CompilerParams(use_tc_tiling_on_sc=False),
    )


def _tc_mask_body(mask_ref, rows_ref, out_ref, dmask_ref):
    m = mask_ref[0]
    r = lax.broadcasted_iota(jnp.int32, (_N, _N), 0)
    c = lax.broadcasted_iota(jnp.int32, (_N, _N), 1)
    diag = jnp.any(jnp.logical_and(m, r == c), axis=1)
    dmask_ref[0] = diag.reshape(1, _N)
    out_ref[0] = jnp.where(diag[:, None], rows_ref[0], 0.0)


_tc_mask = pl.pallas_call(
    _tc_mask_body,
    grid=(_B,),
    in_specs=[
        pl.BlockSpec((1, _N, _N), lambda b: (b, 0, 0)),
        pl.BlockSpec((1, _N, _D), lambda b: (b, 0, 0)),
    ],
    out_specs=[
        pl.BlockSpec((1, _N, _D), lambda b: (b, 0, 0)),
        pl.BlockSpec((1, 1, _N), lambda b: (b, 0, 0)),
    ],
    out_shape=[
        jax.ShapeDtypeStruct((_B, _N, _D), jnp.float32),
        jax.ShapeDtypeStruct((_B, 1, _N), jnp.bool_),
    ],
)


def kernel(X_data, X_mask):
    table = X_data.reshape(_B * _N * _N, _D)
    rows = _sc_gather()(table)
    out_data, dmask = _tc_mask(X_mask, rows.reshape(_B, _N, _D))
    return out_data, dmask.reshape(_B, _N)


# speedup vs baseline: 1.0022x; 1.0022x over previous
import functools

import jax
import jax.numpy as jnp
from jax import lax
from jax.experimental import pallas as pl
from jax.experimental.pallas import tpu as pltpu
from jax.experimental.pallas import tpu_sc as plsc

_B, _N, _D = 16, 512, 16
_NC, _NS = 2, 16
_NW = _NC * _NS
_R = _B * _N
_RPW = _R // _NW
_CH = 128
_W = 128
_V = _B * _N * _N * _D // _W


def _sc_row_ids(idx_ref, start):
    for t in range(_CH // 16):
        g = start + t * 16 + lax.broadcasted_iota(jnp.int32, (16,), 0)
        b = g >> 9
        i = g & (_N - 1)
        idx_ref[pl.ds(t * 16, 16)] = (b << 15) + ((i * (_N + 1)) >> 3)


def _sc_gather_body(table_hbm, out_hbm, idx0, idx1, big0, big1, small, sem):
    wid = lax.axis_index("s") * _NC + lax.axis_index("c")
    base = wid * _RPW
    _sc_row_ids(idx0, base)
    _sc_row_ids(idx1, base + _CH)
    cp0 = pltpu.async_copy(table_hbm.at[idx0], big0, sem)
    cp1 = pltpu.async_copy(table_hbm.at[idx1], big1, sem)
    cp0.wait()
    cp1.wait()
    for j, big in enumerate((big0, big1)):
        for r in range(_CH):
            off = 16 * (r % 8)
            small[pl.ds((j * _CH + r) * _D, _D)] = big[r, pl.ds(off, _D)]
    pltpu.sync_copy(small, out_hbm.at[pl.ds(base * _D, _RPW * _D)])


@functools.cache
def _sc_gather():
    return pl.kernel(
        _sc_gather_body,
        out_type=jax.ShapeDtypeStruct((_R * _D,), jnp.float32),
        mesh=plsc.VectorSubcoreMesh(
            core_axis_name="c", subcore_axis_name="s",
            num_cores=_NC, num_subcores=_NS,
        ),
        scratch_types=[
            pltpu.VMEM((_CH,), jnp.int32),
            pltpu.VMEM((_CH,), jnp.int32),
            pltpu.VMEM((_CH, _W), jnp.float32),
            pltpu.VMEM((_CH, _W), jnp.float32),
            pltpu.VMEM((_RPW * _D,), jnp.float32),
            pltpu.SemaphoreType.DMA,
        ],
    )


def _tc_mask_body(mask_ref, rows_ref, out_ref, dmask_ref):
    m = mask_ref[0]
    r = lax.broadcasted_iota(jnp.int32, (_N, _N), 0)
    c = lax.broadcasted_iota(jnp.int32, (_N, _N), 1)
    diag = jnp.any(jnp.logical_and(m, r == c), axis=1)
    dmask_ref[0] = diag.reshape(1, _N)
    out_ref[0] = jnp.where(diag[:, None], rows_ref[0], 0.0)


_tc_mask = pl.pallas_call(
    _tc_mask_body,
    grid=(_B,),
    in_specs=[
        pl.BlockSpec((1, _N, _N), lambda b: (b, 0, 0)),
        pl.BlockSpec((1, _N, _D), lambda b: (b, 0, 0)),
    ],
    out_specs=[
        pl.BlockSpec((1, _N, _D), lambda b: (b, 0, 0)),
        pl.BlockSpec((1, 1, _N), lambda b: (b, 0, 0)),
    ],
    out_shape=[
        jax.ShapeDtypeStruct((_B, _N, _D), jnp.float32),
        jax.ShapeDtypeStruct((_B, 1, _N), jnp.bool_),
    ],
)


def kernel(X_data, X_mask):
    table = X_data.reshape(_V, _W)
    rows = _sc_gather()(table)
    out_data, dmask = _tc_mask(X_mask, rows.reshape(_B, _N, _D))
    return out_data, dmask.reshape(_B, _N)


# speedup vs baseline: 25.2030x; 25.1483x over previous
import functools

import jax
import jax.numpy as jnp
from jax import lax
from jax.experimental import pallas as pl
from jax.experimental.pallas import tpu as pltpu
from jax.experimental.pallas import tpu_sc as plsc

_B, _N, _D = 16, 512, 16
_NC, _NS = 2, 16
_NW = _NC * _NS
_R = _B * _N
_RPW = _R // _NW
_EPW = _RPW * _D
_CH = 128
_NCH = _EPW // _CH


def _sc_gather_body(table_hbm, out_hbm, svec, idx_v, dst, sem):
    wid = lax.axis_index("s") * _NC + lax.axis_index("c")
    base = wid * _RPW
    bb = base >> 9
    n0 = pl.multiple_of(base & (_N - 1), _RPW)
    lane = lax.broadcasted_iota(jnp.int32, (16,), 0)
    for t in range(_RPW // 16):
        i = n0 + t * 16 + lane
        svec[pl.ds(t * 16, 16)] = (bb << 22) + (i << 13) + ((i >> 7) << 10) + (i & 127)
    for c in range(_NCH):
        d = c // 2
        d_off = ((d >> 3) << 12) + ((d & 7) << 7)
        for t in range(_CH // 16):
            src = (c % 2) * _CH + t * 16
            idx_v[c, pl.ds(t * 16, 16)] = svec[pl.ds(src, 16)] + d_off
    cps = [
        pltpu.async_copy(table_hbm.at[idx_v.at[c]],
                         dst.at[c // 2, pl.ds((c % 2) * _CH, _CH)], sem)
        for c in range(_NCH)
    ]
    for cp in cps:
        cp.wait()
    pltpu.sync_copy(dst, out_hbm.at[bb, :, pl.ds(n0, _RPW)])


@functools.cache
def _sc_gather():
    return pl.kernel(
        _sc_gather_body,
        out_type=jax.ShapeDtypeStruct((_B, _D, _N), jnp.float32),
        mesh=plsc.VectorSubcoreMesh(
            core_axis_name="c", subcore_axis_name="s",
            num_cores=_NC, num_subcores=_NS,
        ),
        scratch_types=[
            pltpu.VMEM((_RPW,), jnp.int32),
            pltpu.VMEM((_NCH, _CH), jnp.int32),
            pltpu.VMEM((_D, _RPW), jnp.float32),
            pltpu.SemaphoreType.DMA,
        ],
        compiler_params=pltpu.CompilerParams(use_tc_tiling_on_sc=False),
    )


_BK = 128


def _tc_mask_body(mask_ref, rows_ref, out_ref, red_ref):
    mb = mask_ref[0].astype(jnp.int32)
    r = lax.broadcasted_iota(jnp.int32, (_BK, _BK), 0)
    c = lax.broadcasted_iota(jnp.int32, (_BK, _BK), 1)
    red = jnp.max(jnp.where(r == c, mb, 0), axis=0)
    red_ref[0] = red.reshape(1, _BK)
    keep = jnp.broadcast_to(red[None, :], (_D, _BK)) != 0
    out_ref[0] = jnp.where(keep, rows_ref[0], 0.0)


_tc_mask = pl.pallas_call(
    _tc_mask_body,
    grid=(_B, _N // _BK),
    in_specs=[
        pl.BlockSpec((1, _BK, _BK), lambda b, j: (b, j, j)),
        pl.BlockSpec((1, _D, _BK), lambda b, j: (b, 0, j)),
    ],
    out_specs=[
        pl.BlockSpec((1, _D, _BK), lambda b, j: (b, 0, j)),
        pl.BlockSpec((1, 1, _BK), lambda b, j: (b, 0, j)),
    ],
    out_shape=[
        jax.ShapeDtypeStruct((_B, _D, _N), jnp.float32),
        jax.ShapeDtypeStruct((_B, 1, _N), jnp.int32),
    ],
)


def kernel(X_data, X_mask):
    table = (
        X_data.reshape(_B, _N, 4, 128, 2, 8)
        .transpose(0, 1, 4, 2, 5, 3)
        .reshape(_B * _N * _N * _D)
    )
    rows_t = _sc_gather()(table)
    out_t, red = _tc_mask(X_mask.astype(jnp.int8), rows_t)
    return out_t.transpose(0, 2, 1), red.reshape(_B, _N).astype(jnp.bool_)
